# Initial kernel scaffold; baseline (speedup 1.0000x reference)
import jax
import jax.numpy as jnp
from jax.experimental import pallas as pl


def kernel(x, gamma1, W_in, conv_w, conv_b, W_gates, b_gates, forget_base, W_out, gamma2, W_grow, W_shrink):
    raise NotImplementedError("write your pallas kernel here")


# speedup vs baseline: 6.3883x; 6.3883x over previous
import functools

import jax
import jax.numpy as jnp
from jax.experimental import pallas as pl
from jax.experimental.pallas import tpu as pltpu

DIM = 1024
HID = 1536
GHID = 2048
K = 4
C = 256
SQRT_DIM = 32.0


def _gelu(x):
    return 0.5 * x * (1.0 + jax.lax.erf(x * 0.7071067811865476))


def _dot_bf16(a, b):
    return jax.lax.dot_general(
        a, b, (((1,), (0,)), ((), ())), preferred_element_type=jnp.float32)


def _dot_split(lmat_bf, x):
    x_hi = x.astype(jnp.bfloat16)
    x_lo = (x - x_hi.astype(jnp.float32)).astype(jnp.bfloat16)
    return _dot_bf16(lmat_bf, x_hi) + _dot_bf16(lmat_bf, x_lo)


def _hawk_kernel(x_ref, gamma1_ref, winT_ref, cw_ref, cb_ref, wgT_ref,
                 bg_ref, fb_ref, woT_ref, out_ref, vcarry, hcarry):
    it = pl.program_id(1)

    @pl.when(it == 0)
    def _init():
        vcarry[...] = jnp.zeros_like(vcarry)
        hcarry[...] = jnp.zeros_like(hcarry)

    x = x_ref[0]
    ss = jnp.sum(x * x, axis=1, keepdims=True)
    xn = (x * jax.lax.rsqrt(ss)) * (gamma1_ref[...] * SQRT_DIM)
    u = _dot_bf16(xn.astype(jnp.bfloat16), winT_ref[...])
    gate = u[:, :HID]
    v0 = u[:, HID:]

    prev = vcarry[5:8]
    vm1 = jnp.concatenate([prev[2:3], v0[:-1]], axis=0)
    vm2 = jnp.concatenate([prev[1:3], v0[:-2]], axis=0)
    vm3 = jnp.concatenate([prev[0:3], v0[:-3]], axis=0)
    vcarry[5:8] = v0[C - 3:]
    w = cw_ref
    v = (v0 * w[3:4] + vm1 * w[2:3] + vm2 * w[1:2] + vm3 * w[0:1]
         + cb_ref[...])

    g = _dot_bf16(v.astype(jnp.bfloat16), wgT_ref[...]) + bg_ref[...]
    forget = g[:, :HID]
    inp = g[:, HID:]

    sp = jnp.log1p(jnp.exp(fb_ref[...]))
    la = (-8.0 * sp) * jax.nn.sigmoid(forget)
    a2 = jnp.exp(2.0 * la)
    beta = jnp.sqrt(1.0 - a2 + 1e-6)
    xg = beta * jax.nn.sigmoid(inp) * v

    row = jax.lax.broadcasted_iota(jnp.int32, (C, C), 0)
    col = jax.lax.broadcasted_iota(jnp.int32, (C, C), 1)
    lmat = jnp.where(row >= col, 1.0, 0.0).astype(jnp.bfloat16)

    cl = _dot_split(lmat, la)
    e1 = jnp.exp(cl)
    e = xg * jnp.exp(-cl)
    s = _dot_split(lmat, e)
    h_in = hcarry[0:1]
    h = e1 * (h_in + s)
    hcarry[0:1] = h[C - 1:]

    go = _gelu(gate) * h
    ho = _dot_bf16(go.astype(jnp.bfloat16), woT_ref[...])
    out_ref[0] = x + ho


def _mlp_kernel(x_ref, gamma2_ref, wgrT_ref, wsT_ref, out_ref):
    x = x_ref[0]
    ss = jnp.sum(x * x, axis=1, keepdims=True)
    xn = (x * jax.lax.rsqrt(ss)) * (gamma2_ref[...] * SQRT_DIM)
    grow = _dot_bf16(xn.astype(jnp.bfloat16), wgrT_ref[...])
    m = _gelu(grow[:, :GHID]) * grow[:, GHID:]
    o = _dot_bf16(m.astype(jnp.bfloat16), wsT_ref[...])
    out_ref[0] = x + o


@functools.partial(jax.jit, static_argnames=())
def kernel(x, gamma1, W_in, conv_w, conv_b, W_gates, b_gates, forget_base,
           W_out, gamma2, W_grow, W_shrink):
    n, t, d = x.shape
    nt = t // C
    f32 = jnp.float32
    bf16 = jnp.bfloat16

    winT = W_in.T.astype(bf16)
    wgT = W_gates.T.astype(bf16)
    woT = W_out.T.astype(bf16)
    wgrT = W_grow.T.astype(bf16)
    wsT = W_shrink.T.astype(bf16)
    cw = jnp.pad(conv_w[:, 0, :].T, ((0, 4), (0, 0)))
    cb = conv_b.reshape(1, HID)
    bg = b_gates.reshape(1, 2 * HID)
    fb = forget_base.reshape(1, HID)
    g1 = gamma1.reshape(1, d)
    g2 = gamma2.reshape(1, d)

    full = lambda s: pl.BlockSpec(s, lambda i, j: (0,) * len(s))

    xh = pl.pallas_call(
        _hawk_kernel,
        grid=(n, nt),
        in_specs=[
            pl.BlockSpec((1, C, d), lambda i, j: (i, j, 0)),
            full((1, d)),
            full((d, 2 * HID)),
            full((8, HID)),
            full((1, HID)),
            full((HID, 2 * HID)),
            full((1, 2 * HID)),
            full((1, HID)),
            full((HID, d)),
        ],
        out_specs=pl.BlockSpec((1, C, d), lambda i, j: (i, j, 0)),
        out_shape=jax.ShapeDtypeStruct((n, t, d), f32),
        scratch_shapes=[
            pltpu.VMEM((8, HID), f32),
            pltpu.VMEM((8, HID), f32),
        ],
        compiler_params=pltpu.CompilerParams(
            dimension_semantics=("parallel", "arbitrary"),
            vmem_limit_bytes=56 * 1024 * 1024,
        ),
    )(x, g1, winT, cw, cb, wgT, bg, fb, woT)

    xb = xh.reshape(n * nt, C, d)
    out = pl.pallas_call(
        _mlp_kernel,
        grid=(n * nt,),
        in_specs=[
            pl.BlockSpec((1, C, d), lambda i: (i, 0, 0)),
            pl.BlockSpec((1, d), lambda i: (0, 0)),
            pl.BlockSpec((d, 2 * GHID), lambda i: (0, 0)),
            pl.BlockSpec((GHID, d), lambda i: (0, 0)),
        ],
        out_specs=pl.BlockSpec((1, C, d), lambda i: (i, 0, 0)),
        out_shape=jax.ShapeDtypeStruct((n * nt, C, d), f32),
        compiler_params=pltpu.CompilerParams(
            dimension_semantics=("parallel",),
            vmem_limit_bytes=56 * 1024 * 1024,
        ),
    )(xb, g2, wgrT, wsT)
    return out.reshape(n, t, d)


# speedup vs baseline: 6.7263x; 1.0529x over previous
import functools

import jax
import jax.numpy as jnp
from jax.experimental import pallas as pl
from jax.experimental.pallas import tpu as pltpu

DIM = 1024
HID = 1536
GHID = 2048
K = 4
C = 256
SQRT_DIM = 32.0


def _gelu(x):
    return 0.5 * x * (1.0 + jax.lax.erf(x * 0.7071067811865476))


def _dot_bf16(a, b):
    return jax.lax.dot_general(
        a, b, (((1,), (0,)), ((), ())), preferred_element_type=jnp.float32)


def _dot_split(lmat_bf, x):
    x_hi = x.astype(jnp.bfloat16)
    x_lo = (x - x_hi.astype(jnp.float32)).astype(jnp.bfloat16)
    return _dot_bf16(lmat_bf, x_hi) + _dot_bf16(lmat_bf, x_lo)


def _rmsnorm(x, gamma_row):
    ss = jnp.sum(x * x, axis=1, keepdims=True)
    return (x * jax.lax.rsqrt(ss)) * (gamma_row * SQRT_DIM)


def _hawk_chunk(x, gamma1_ref, winT_ref, cw_ref, cb_ref, wgT_ref, bg_ref,
                fb_ref, woT_ref, lmat, smats, vcarry, hcarry):
    xn = _rmsnorm(x, gamma1_ref[...])
    u = _dot_bf16(xn.astype(jnp.bfloat16), winT_ref[...])
    gate = u[:, :HID]
    v0 = u[:, HID:]

    v0b = v0.astype(jnp.bfloat16)
    prev = vcarry[5:8]
    f8_1 = jnp.concatenate([prev[2:3], v0[0:7]], axis=0)
    f8_2 = jnp.concatenate([prev[1:3], v0[0:6]], axis=0)
    f8_3 = jnp.concatenate([prev[0:3], v0[0:5]], axis=0)
    vm1 = jnp.concatenate([f8_1, _dot_bf16(smats[0], v0b)[8:]], axis=0)
    vm2 = jnp.concatenate([f8_2, _dot_bf16(smats[1], v0b)[8:]], axis=0)
    vm3 = jnp.concatenate([f8_3, _dot_bf16(smats[2], v0b)[8:]], axis=0)
    vcarry[5:8] = v0[C - 3:]
    w = cw_ref
    v = (v0 * w[3:4] + vm1 * w[2:3] + vm2 * w[1:2] + vm3 * w[0:1]
         + cb_ref[...])

    g = _dot_bf16(v.astype(jnp.bfloat16), wgT_ref[...]) + bg_ref[...]
    forget = g[:, :HID]
    inp = g[:, HID:]

    sp = jnp.log1p(jnp.exp(fb_ref[...]))
    la = (-8.0 * sp) * jax.nn.sigmoid(forget)
    a2 = jnp.exp(2.0 * la)
    beta = jnp.sqrt(1.0 - a2 + 1e-6)
    xg = beta * jax.nn.sigmoid(inp) * v

    cl = _dot_split(lmat, la)
    e1 = jnp.exp(cl)
    e = xg / e1
    s = _dot_split(lmat, e)
    h_in = hcarry[0:1]
    h = e1 * (h_in + s)
    hcarry[0:1] = h[C - 1:]

    go = _gelu(gate) * h
    ho = _dot_bf16(go.astype(jnp.bfloat16), woT_ref[...])
    return x + ho


def _hawk_kernel(x_ref, gamma1_ref, winT_ref, cw_ref, cb_ref, wgT_ref,
                 bg_ref, fb_ref, woT_ref, out_ref, vcarry, hcarry):
    it = pl.program_id(1)

    @pl.when(it == 0)
    def _init():
        vcarry[...] = jnp.zeros_like(vcarry)
        hcarry[...] = jnp.zeros_like(hcarry)

    row = jax.lax.broadcasted_iota(jnp.int32, (C, C), 0)
    col = jax.lax.broadcasted_iota(jnp.int32, (C, C), 1)
    lmat = jnp.where(row >= col, 1.0, 0.0).astype(jnp.bfloat16)
    smats = [jnp.where(row - col == k, 1.0, 0.0).astype(jnp.bfloat16)
             for k in (1, 2, 3)]

    for g in range(2):
        out_ref[g] = _hawk_chunk(
            x_ref[g], gamma1_ref, winT_ref, cw_ref, cb_ref, wgT_ref, bg_ref,
            fb_ref, woT_ref, lmat, smats, vcarry.at[g], hcarry.at[g])


def _mlp_block(x, gamma2_ref, wgrT_ref, wsT_ref):
    xn = _rmsnorm(x, gamma2_ref[...])
    grow = _dot_bf16(xn.astype(jnp.bfloat16), wgrT_ref[...])
    m = _gelu(grow[:, :GHID]) * grow[:, GHID:]
    o = _dot_bf16(m.astype(jnp.bfloat16), wsT_ref[...])
    return x + o


def _mlp_kernel(x_ref, gamma2_ref, wgrT_ref, wsT_ref, out_ref):
    for g in range(2):
        out_ref[g] = _mlp_block(x_ref[g], gamma2_ref, wgrT_ref, wsT_ref)


@functools.partial(jax.jit, static_argnames=())
def kernel(x, gamma1, W_in, conv_w, conv_b, W_gates, b_gates, forget_base,
           W_out, gamma2, W_grow, W_shrink):
    n, t, d = x.shape
    nt = t // C
    f32 = jnp.float32
    bf16 = jnp.bfloat16

    winT = W_in.T.astype(bf16)
    wgT = W_gates.T.astype(bf16)
    woT = W_out.T.astype(bf16)
    wgrT = W_grow.T.astype(bf16)
    wsT = W_shrink.T.astype(bf16)
    cw = jnp.pad(conv_w[:, 0, :].T, ((0, 4), (0, 0)))
    cb = conv_b.reshape(1, HID)
    bg = b_gates.reshape(1, 2 * HID)
    fb = forget_base.reshape(1, HID)
    g1 = gamma1.reshape(1, d)
    g2 = gamma2.reshape(1, d)

    full = lambda s: pl.BlockSpec(s, lambda i, j: (0,) * len(s))

    xh = pl.pallas_call(
        _hawk_kernel,
        grid=(n // 2, nt),
        in_specs=[
            pl.BlockSpec((2, C, d), lambda i, j: (i, j, 0)),
            full((1, d)),
            full((d, 2 * HID)),
            full((8, HID)),
            full((1, HID)),
            full((HID, 2 * HID)),
            full((1, 2 * HID)),
            full((1, HID)),
            full((HID, d)),
        ],
        out_specs=pl.BlockSpec((2, C, d), lambda i, j: (i, j, 0)),
        out_shape=jax.ShapeDtypeStruct((n, t, d), f32),
        scratch_shapes=[
            pltpu.VMEM((2, 8, HID), f32),
            pltpu.VMEM((2, 8, HID), f32),
        ],
        compiler_params=pltpu.CompilerParams(
            dimension_semantics=("arbitrary", "arbitrary"),
            vmem_limit_bytes=56 * 1024 * 1024,
        ),
    )(x, g1, winT, cw, cb, wgT, bg, fb, woT)

    xb = xh.reshape(n * nt, C, d)
    out = pl.pallas_call(
        _mlp_kernel,
        grid=(n * nt // 2,),
        in_specs=[
            pl.BlockSpec((2, C, d), lambda i: (i, 0, 0)),
            pl.BlockSpec((1, d), lambda i: (0, 0)),
            pl.BlockSpec((d, 2 * GHID), lambda i: (0, 0)),
            pl.BlockSpec((GHID, d), lambda i: (0, 0)),
        ],
        out_specs=pl.BlockSpec((2, C, d), lambda i: (i, 0, 0)),
        out_shape=jax.ShapeDtypeStruct((n * nt, C, d), f32),
        compiler_params=pltpu.CompilerParams(
            dimension_semantics=("arbitrary",),
            vmem_limit_bytes=56 * 1024 * 1024,
        ),
    )(xb, g2, wgrT, wsT)
    return out.reshape(n, t, d)


# speedup vs baseline: 6.9201x; 1.0288x over previous
import functools

import jax
import jax.numpy as jnp
from jax.experimental import pallas as pl
from jax.experimental.pallas import tpu as pltpu

DIM = 1024
HID = 1536
GHID = 2048
K = 4
C = 256
SQRT_DIM = 32.0


def _gelu(x):
    return 0.5 * x * (1.0 + jax.lax.erf(x * 0.7071067811865476))


def _dot_bf16(a, b):
    return jax.lax.dot_general(
        a, b, (((1,), (0,)), ((), ())), preferred_element_type=jnp.float32)


def _rmsnorm(x, gamma_row):
    ss = jnp.sum(x * x, axis=1, keepdims=True)
    return (x * jax.lax.rsqrt(ss)) * (gamma_row * SQRT_DIM)


def _hawk_chunk(x, gamma1_ref, winT_ref, cw_ref, cb_ref, wgT_ref, bg_ref,
                fb_ref, woT_ref, lmat, smats, vcarry, hcarry):
    xn = _rmsnorm(x, gamma1_ref[...])
    u = _dot_bf16(xn.astype(jnp.bfloat16), winT_ref[...])
    gate = u[:, :HID]
    v0 = u[:, HID:]

    v0b = v0.astype(jnp.bfloat16)
    prev = vcarry[5:8]
    f8_1 = jnp.concatenate([prev[2:3], v0[0:7]], axis=0)
    f8_2 = jnp.concatenate([prev[1:3], v0[0:6]], axis=0)
    f8_3 = jnp.concatenate([prev[0:3], v0[0:5]], axis=0)
    vm1 = jnp.concatenate([f8_1, _dot_bf16(smats[0], v0b)[8:]], axis=0)
    vm2 = jnp.concatenate([f8_2, _dot_bf16(smats[1], v0b)[8:]], axis=0)
    vm3 = jnp.concatenate([f8_3, _dot_bf16(smats[2], v0b)[8:]], axis=0)
    vcarry[5:8] = v0[C - 3:]
    w = cw_ref
    v = (v0 * w[3:4] + vm1 * w[2:3] + vm2 * w[1:2] + vm3 * w[0:1]
         + cb_ref[...])

    g = _dot_bf16(v.astype(jnp.bfloat16), wgT_ref[...]) + bg_ref[...]
    forget = g[:, :HID]
    inp = g[:, HID:]

    sp = jnp.log1p(jnp.exp(fb_ref[...]))
    la = (-8.0 * sp) * jax.nn.sigmoid(forget)
    a2 = jnp.exp(2.0 * la)
    beta = jnp.sqrt(1.0 - a2 + 1e-6)
    xg = beta * jax.nn.sigmoid(inp) * v

    cl = _dot_bf16(lmat, la.astype(jnp.bfloat16))
    e1 = jnp.exp(cl)
    s = _dot_bf16(lmat, (xg / e1).astype(jnp.bfloat16))
    h_in = hcarry[0:1]
    h = e1 * (h_in + s)
    hcarry[0:1] = h[C - 1:]

    go = _gelu(gate) * h
    ho = _dot_bf16(go.astype(jnp.bfloat16), woT_ref[...])
    return x + ho


def _hawk_kernel(x_ref, gamma1_ref, winT_ref, cw_ref, cb_ref, wgT_ref,
                 bg_ref, fb_ref, woT_ref, out_ref, vcarry, hcarry):
    it = pl.program_id(1)

    @pl.when(it == 0)
    def _init():
        vcarry[...] = jnp.zeros_like(vcarry)
        hcarry[...] = jnp.zeros_like(hcarry)

    row = jax.lax.broadcasted_iota(jnp.int32, (C, C), 0)
    col = jax.lax.broadcasted_iota(jnp.int32, (C, C), 1)
    lmat = jnp.where(row >= col, 1.0, 0.0).astype(jnp.bfloat16)
    smats = [jnp.where(row - col == k, 1.0, 0.0).astype(jnp.bfloat16)
             for k in (1, 2, 3)]

    for g in range(2):
        out_ref[g] = _hawk_chunk(
            x_ref[g], gamma1_ref, winT_ref, cw_ref, cb_ref, wgT_ref, bg_ref,
            fb_ref, woT_ref, lmat, smats, vcarry.at[g], hcarry.at[g])


def _mlp_block(x, gamma2_ref, wgrT_ref, wsT_ref):
    xn = _rmsnorm(x, gamma2_ref[...])
    grow = _dot_bf16(xn.astype(jnp.bfloat16), wgrT_ref[...])
    m = _gelu(grow[:, :GHID]) * grow[:, GHID:]
    o = _dot_bf16(m.astype(jnp.bfloat16), wsT_ref[...])
    return x + o


def _mlp_kernel(x_ref, gamma2_ref, wgrT_ref, wsT_ref, out_ref):
    for g in range(2):
        out_ref[g] = _mlp_block(x_ref[g], gamma2_ref, wgrT_ref, wsT_ref)


@functools.partial(jax.jit, static_argnames=())
def kernel(x, gamma1, W_in, conv_w, conv_b, W_gates, b_gates, forget_base,
           W_out, gamma2, W_grow, W_shrink):
    n, t, d = x.shape
    nt = t // C
    f32 = jnp.float32
    bf16 = jnp.bfloat16

    winT = W_in.T.astype(bf16)
    wgT = W_gates.T.astype(bf16)
    woT = W_out.T.astype(bf16)
    wgrT = W_grow.T.astype(bf16)
    wsT = W_shrink.T.astype(bf16)
    cw = jnp.pad(conv_w[:, 0, :].T, ((0, 4), (0, 0)))
    cb = conv_b.reshape(1, HID)
    bg = b_gates.reshape(1, 2 * HID)
    fb = forget_base.reshape(1, HID)
    g1 = gamma1.reshape(1, d)
    g2 = gamma2.reshape(1, d)

    full = lambda s: pl.BlockSpec(s, lambda i, j: (0,) * len(s))

    xh = pl.pallas_call(
        _hawk_kernel,
        grid=(n // 2, nt),
        in_specs=[
            pl.BlockSpec((2, C, d), lambda i, j: (i, j, 0)),
            full((1, d)),
            full((d, 2 * HID)),
            full((8, HID)),
            full((1, HID)),
            full((HID, 2 * HID)),
            full((1, 2 * HID)),
            full((1, HID)),
            full((HID, d)),
        ],
        out_specs=pl.BlockSpec((2, C, d), lambda i, j: (i, j, 0)),
        out_shape=jax.ShapeDtypeStruct((n, t, d), f32),
        scratch_shapes=[
            pltpu.VMEM((2, 8, HID), f32),
            pltpu.VMEM((2, 8, HID), f32),
        ],
        compiler_params=pltpu.CompilerParams(
            dimension_semantics=("arbitrary", "arbitrary"),
            vmem_limit_bytes=56 * 1024 * 1024,
        ),
    )(x, g1, winT, cw, cb, wgT, bg, fb, woT)

    xb = xh.reshape(n * nt, C, d)
    out = pl.pallas_call(
        _mlp_kernel,
        grid=(n * nt // 2,),
        in_specs=[
            pl.BlockSpec((2, C, d), lambda i: (i, 0, 0)),
            pl.BlockSpec((1, d), lambda i: (0, 0)),
            pl.BlockSpec((d, 2 * GHID), lambda i: (0, 0)),
            pl.BlockSpec((GHID, d), lambda i: (0, 0)),
        ],
        out_specs=pl.BlockSpec((2, C, d), lambda i: (i, 0, 0)),
        out_shape=jax.ShapeDtypeStruct((n * nt, C, d), f32),
        compiler_params=pltpu.CompilerParams(
            dimension_semantics=("arbitrary",),
            vmem_limit_bytes=56 * 1024 * 1024,
        ),
    )(xb, g2, wgrT, wsT)
    return out.reshape(n, t, d)
